# baseline (speedup 1.0000x reference)
import jax
import jax.numpy as jnp
from jax import lax
from jax.experimental import pallas as pl
from jax.experimental.pallas import tpu as pltpu
from jax.experimental.pallas import tpu_sc as plsc

_N = 32768
_E = 64
_K = 8
_L = 16
_NW = 32
_PER_W = _N // _NW
_CH = 256
_NCHUNK = _PER_W // _CH
_GROUPS = _CH // _L
_VS = 17


def _tournament(leaves):
    while len(leaves) > 1:
        nxt = []
        for p in range(0, len(leaves), 2):
            (va, ia), (vb, ib) = leaves[p], leaves[p + 1]
            c = va >= vb
            nxt.append((jnp.where(c, va, vb), jnp.where(c, ia, ib)))
        leaves = nxt
    return leaves[0]


def _sc_body(x_hbm, bias_hbm, idx_hbm, w_hbm, xbuf, vt, idxbuf, wbuf, biasv):
    nc = plsc.get_sparse_core_info().num_cores
    wid = lax.axis_index("s") * nc + lax.axis_index("c")
    lane = jnp.arange(_L, dtype=jnp.int32)
    neg = jnp.full((_L,), -jnp.inf, dtype=jnp.float32)
    lane17 = lane * _VS

    pltpu.sync_copy(bias_hbm, biasv)
    bvecs = [biasv[pl.ds(u * _L, _L)] for u in range(_E // _L)]

    def chunk_body(c, carry):
        base = wid * _PER_W + c * _CH
        pltpu.sync_copy(x_hbm.at[pl.ds(base, _CH)], xbuf)

        def group_body(g, carry2):
            row = g * _L + lane
            for t in range(_L):
                tok = g * _L + t
                for u in range(_E // _L):
                    v = xbuf[tok, pl.ds(u * _L, _L)] + bvecs[u]
                    plsc.store_scatter(
                        vt, [lane17 + (_VS * _L * u + t)], v)
            vals, idxs = [], []
            for r in range(_K):
                subroots = []
                for s in range(4):
                    leaves = [(vt[pl.ds((s * 16 + j) * _VS, _L)],
                               jnp.full((_L,), s * 16 + j, dtype=jnp.int32))
                              for j in range(16)]
                    subroots.append(_tournament(leaves))
                m, am = _tournament(subroots)
                vals.append(m)
                idxs.append(am)
                if r < _K - 1:
                    plsc.store_scatter(vt, [am * _VS + lane], neg)
            es = [jnp.exp(v - vals[0]) for v in vals]
            ssum = es[0]
            for t in es[1:]:
                ssum = ssum + t
            rinv = 1.0 / ssum
            for r in range(_K):
                colr = jnp.full((_L,), r, dtype=jnp.int32)
                plsc.store_scatter(idxbuf, [row, colr], idxs[r])
                plsc.store_scatter(wbuf, [row, colr], es[r] * rinv)
            return carry2

        lax.fori_loop(0, _GROUPS, group_body, 0)
        pltpu.sync_copy(idxbuf, idx_hbm.at[pl.ds(base, _CH)])
        pltpu.sync_copy(wbuf, w_hbm.at[pl.ds(base, _CH)])
        return carry

    lax.fori_loop(0, _NCHUNK, chunk_body, 0)


@jax.jit
def kernel(gate_logits, bias):
    mesh = plsc.VectorSubcoreMesh(core_axis_name="c", subcore_axis_name="s")
    run = pl.kernel(
        _sc_body,
        out_type=[
            jax.ShapeDtypeStruct((_N, _K), jnp.int32),
            jax.ShapeDtypeStruct((_N, _K), jnp.float32),
        ],
        mesh=mesh,
        compiler_params=pltpu.CompilerParams(needs_layout_passes=False),
        scratch_types=[
            pltpu.VMEM((_CH, _E), jnp.float32),
            pltpu.VMEM((_E * _VS,), jnp.float32),
            pltpu.VMEM((_CH, _K), jnp.int32),
            pltpu.VMEM((_CH, _K), jnp.float32),
            pltpu.VMEM((_E,), jnp.float32),
        ],
    )
    idx, w = run(gate_logits, bias)
    return idx, w


# speedup vs baseline: 1.6278x; 1.6278x over previous
import jax
import jax.numpy as jnp
from jax import lax
from jax.experimental import pallas as pl
from jax.experimental.pallas import tpu as pltpu
from jax.experimental.pallas import tpu_sc as plsc

_N = 32768
_E = 64
_K = 8
_L = 16
_NW = 32
_PER_W = _N // _NW
_CH = 256
_NCHUNK = _PER_W // _CH
_GROUPS = _CH // _L
_ABLK = _CH * _K
_ASTR = _K * _N


def _tournament(leaves):
    while len(leaves) > 1:
        nxt = []
        for p in range(0, len(leaves), 2):
            (va, ia), (vb, ib) = leaves[p], leaves[p + 1]
            c = va >= vb
            nxt.append((jnp.where(c, va, vb), jnp.where(c, ia, ib)))
        leaves = nxt
    return leaves[0]


def _sc_body(x_hbm, bias_hbm, idx_hbm, w_hbm, xbuf, oibuf, owbuf, biasv):
    nc = plsc.get_sparse_core_info().num_cores
    wid = lax.axis_index("s") * nc + lax.axis_index("c")
    lane = jnp.arange(_L, dtype=jnp.int32)
    neg = jnp.full((_L,), -jnp.inf, dtype=jnp.float32)

    pltpu.sync_copy(bias_hbm, biasv)
    bvals = []
    for s in range(_E // _L):
        bvec = biasv[pl.ds(s * _L, _L)]
        bvals.extend(bvec[j] for j in range(_L))

    def chunk_body(c, carry):
        t0 = wid * _PER_W + c * _CH
        for a in range(_E // _K):
            pltpu.sync_copy(x_hbm.at[pl.ds(a * _ASTR + t0 * _K, _ABLK)],
                            xbuf.at[pl.ds(a * _ABLK, _ABLK)])

        def group_body(g, carry2):
            bg = (g >> 3) * 1024 + (g & 7) * _L
            vals, idxs = [], []
            for r in range(_K):
                subroots = []
                for s in range(4):
                    leaves = []
                    for j in range(16):
                        e = s * 16 + j
                        off = (e >> 3) * _ABLK + (e & 7) * 128
                        v = xbuf[pl.ds(bg + off, _L)]
                        if r == 0:
                            v = v + bvals[e]
                        leaves.append(
                            (v, jnp.full((_L,), e, dtype=jnp.int32)))
                    subroots.append(_tournament(leaves))
                m, am = _tournament(subroots)
                vals.append(m)
                idxs.append(am)
                if r == 0:
                    for e in range(_E):
                        off = (e >> 3) * _ABLK + (e & 7) * 128
                        ref = xbuf.at[pl.ds(bg + off, _L)]
                        ref[...] = ref[...] + bvals[e]
                if r < _K - 1:
                    pos = ((am >> 3) * _ABLK + (am & 7) * 128 + bg) + lane
                    plsc.store_scatter(xbuf, [pos], neg)
            es = [jnp.exp(v - vals[0]) for v in vals]
            ssum = es[0]
            for t in es[1:]:
                ssum = ssum + t
            rinv = 1.0 / ssum
            for r in range(_K):
                oibuf[pl.ds(bg + r * 128, _L)] = idxs[r]
                owbuf[pl.ds(bg + r * 128, _L)] = es[r] * rinv
            return carry2

        lax.fori_loop(0, _GROUPS, group_body, 0)
        pltpu.sync_copy(oibuf, idx_hbm.at[pl.ds(t0 * _K, _ABLK)])
        pltpu.sync_copy(owbuf, w_hbm.at[pl.ds(t0 * _K, _ABLK)])
        return carry

    lax.fori_loop(0, _NCHUNK, chunk_body, 0)


@jax.jit
def kernel(gate_logits, bias):
    xf = (gate_logits.reshape(_N // 128, 128, _E // _K, _K)
          .transpose(2, 0, 3, 1)
          .reshape(_N * _E))
    mesh = plsc.VectorSubcoreMesh(core_axis_name="c", subcore_axis_name="s")
    run = pl.kernel(
        _sc_body,
        out_type=[
            jax.ShapeDtypeStruct((_N * _K,), jnp.int32),
            jax.ShapeDtypeStruct((_N * _K,), jnp.float32),
        ],
        mesh=mesh,
        compiler_params=pltpu.CompilerParams(needs_layout_passes=False),
        scratch_types=[
            pltpu.VMEM((_E * _CH,), jnp.float32),
            pltpu.VMEM((_ABLK,), jnp.int32),
            pltpu.VMEM((_ABLK,), jnp.float32),
            pltpu.VMEM((_E,), jnp.float32),
        ],
    )
    idxf, wf = run(xf, bias)
    idx = idxf.reshape(_N // 128, _K, 128).transpose(0, 2, 1).reshape(_N, _K)
    w = wf.reshape(_N // 128, _K, 128).transpose(0, 2, 1).reshape(_N, _K)
    return idx, w


# speedup vs baseline: 1.9967x; 1.2266x over previous
import jax
import jax.numpy as jnp
from jax import lax
from jax.experimental import pallas as pl
from jax.experimental.pallas import tpu as pltpu
from jax.experimental.pallas import tpu_sc as plsc

_N = 32768
_E = 64
_K = 8
_L = 16
_NW = 32
_PER_W = _N // _NW
_GROUPS = _PER_W // _L
_ABLK = _PER_W * _K
_ASTR = _K * _N


def _tournament(leaves):
    while len(leaves) > 1:
        nxt = []
        for p in range(0, len(leaves), 2):
            (va, ia), (vb, ib) = leaves[p], leaves[p + 1]
            c = va >= vb
            nxt.append((jnp.where(c, va, vb), jnp.where(c, ia, ib)))
        leaves = nxt
    return leaves[0]


def _sc_body(x_hbm, bias_hbm, idx_hbm, w_hbm, xbuf, oibuf, owbuf, biasv, sem):
    nc = plsc.get_sparse_core_info().num_cores
    wid = lax.axis_index("s") * nc + lax.axis_index("c")
    lane = jnp.arange(_L, dtype=jnp.int32)
    neg = jnp.full((_L,), -jnp.inf, dtype=jnp.float32)
    t0 = wid * _PER_W

    descs = [
        pltpu.async_copy(x_hbm.at[pl.ds(a * _ASTR + t0 * _K, _ABLK)],
                         xbuf.at[pl.ds(a * _ABLK, _ABLK)], sem)
        for a in range(_E // _K)
    ]
    pltpu.sync_copy(bias_hbm, biasv)
    bvals = []
    for s in range(_E // _L):
        bvec = biasv[pl.ds(s * _L, _L)]
        bvals.extend(bvec[j] for j in range(_L))
    for d in descs:
        d.wait()

    def group_body(g, carry):
        bg = (g >> 3) * 1024 + (g & 7) * _L
        vals, idxs = [], []
        for r in range(_K):
            subroots = []
            for s in range(4):
                leaves = []
                for j in range(16):
                    e = s * 16 + j
                    off = (e >> 3) * _ABLK + (e & 7) * 128
                    if r == 0:
                        ref = xbuf.at[pl.ds(bg + off, _L)]
                        v = ref[...] + bvals[e]
                        ref[...] = v
                    else:
                        v = xbuf[pl.ds(bg + off, _L)]
                    leaves.append((v, jnp.full((_L,), e, dtype=jnp.int32)))
                subroots.append(_tournament(leaves))
            m, am = _tournament(subroots)
            vals.append(m)
            idxs.append(am)
            if r < _K - 1:
                pos = ((am >> 3) * _ABLK + (am & 7) * 128 + bg) + lane
                plsc.store_scatter(xbuf, [pos], neg)
        es = [jnp.exp(v - vals[0]) for v in vals]
        ssum = es[0]
        for t in es[1:]:
            ssum = ssum + t
        rinv = 1.0 / ssum
        obg = (g >> 3) * 1024 + (g & 7) * _L
        for r in range(_K):
            oibuf[pl.ds(obg + r * 128, _L)] = idxs[r]
            owbuf[pl.ds(obg + r * 128, _L)] = es[r] * rinv
        return carry

    lax.fori_loop(0, _GROUPS, group_body, 0)
    pltpu.sync_copy(oibuf, idx_hbm.at[pl.ds(t0 * _K, _ABLK)])
    pltpu.sync_copy(owbuf, w_hbm.at[pl.ds(t0 * _K, _ABLK)])


@jax.jit
def kernel(gate_logits, bias):
    xf = (gate_logits.reshape(_N // 128, 128, _E // _K, _K)
          .transpose(2, 0, 3, 1)
          .reshape(_N * _E))
    mesh = plsc.VectorSubcoreMesh(core_axis_name="c", subcore_axis_name="s")
    run = pl.kernel(
        _sc_body,
        out_type=[
            jax.ShapeDtypeStruct((_N * _K,), jnp.int32),
            jax.ShapeDtypeStruct((_N * _K,), jnp.float32),
        ],
        mesh=mesh,
        compiler_params=pltpu.CompilerParams(needs_layout_passes=False),
        scratch_types=[
            pltpu.VMEM((_E * _PER_W,), jnp.float32),
            pltpu.VMEM((_ABLK,), jnp.int32),
            pltpu.VMEM((_ABLK,), jnp.float32),
            pltpu.VMEM((_E,), jnp.float32),
            pltpu.SemaphoreType.DMA,
        ],
    )
    idxf, wf = run(xf, bias)
    idx = idxf.reshape(_N // 128, _K, 128).transpose(0, 2, 1).reshape(_N, _K)
    w = wf.reshape(_N // 128, _K, 128).transpose(0, 2, 1).reshape(_N, _K)
    return idx, w
